# Initial kernel scaffold; baseline (speedup 1.0000x reference)
import functools

import jax
import jax.numpy as jnp
from jax import lax
from jax.experimental import pallas as pl
from jax.experimental.pallas import tpu as pltpu
from jax.experimental.pallas import tpu_sc as plsc

N = 10000
E = 320000
D = 128
H = 64
NH = 4
LIN = 128
OUT = 128
ALPHA = 0.2

NC = 2
NS = 16
LANES = 16
NW = NC * NS
EW = E // NW
C = 128
NFULL = EW // C
REM = EW - NFULL * C
ROWS_PER_TILE = N // NS


def _leaky(u):
    return jnp.where(u > 0, u, ALPHA * u)


R1 = 1000


def _p1_body(x_ref, wc_ref, a2_ref, h0, h1, h2, h3, e_ref):
    xb = x_ref[...]
    houts = (h0, h1, h2, h3)
    for k in range(NH):
        hk = jnp.dot(xb, wc_ref[:, k * H:(k + 1) * H],
                     preferred_element_type=jnp.float32)
        houts[k][...] = hk
        ek = lax.dot_general(a2_ref[k], hk, (((1,), (1,)), ((), ())),
                             preferred_element_type=jnp.float32)
        e_ref[:, k, :] = ek


def _phase1(x, w_cat, a2):
    return pl.pallas_call(
        _p1_body,
        grid=(N // R1,),
        in_specs=[
            pl.BlockSpec((R1, D), lambda i: (i, 0)),
            pl.BlockSpec((D, NH * H), lambda i: (0, 0)),
            pl.BlockSpec((NH, 2, H), lambda i: (0, 0, 0)),
        ],
        out_specs=[
            pl.BlockSpec((R1, H), lambda i: (i, 0)),
            pl.BlockSpec((R1, H), lambda i: (i, 0)),
            pl.BlockSpec((R1, H), lambda i: (i, 0)),
            pl.BlockSpec((R1, H), lambda i: (i, 0)),
            pl.BlockSpec((2, NH, R1), lambda i: (0, 0, i)),
        ],
        out_shape=[
            jax.ShapeDtypeStruct((N, H), jnp.float32),
            jax.ShapeDtypeStruct((N, H), jnp.float32),
            jax.ShapeDtypeStruct((N, H), jnp.float32),
            jax.ShapeDtypeStruct((N, H), jnp.float32),
            jax.ShapeDtypeStruct((2, NH, N), jnp.float32),
        ],
    )(x, w_cat, a2)


def _sc_body(h0, h1, h2, h3, src_hbm, dst_hbm, e_hbm,
             acc_out, den_out,
             eS_v, eD_v, den_v, src_v, dst_v, rows_v, ex_v, zbuf,
             src16, dst16, acc_sh, sem):
    c = lax.axis_index("c")
    s = lax.axis_index("s")
    wid = s * NC + c
    hks = (h0, h1, h2, h3)

    def _zrow(r, _):
        for q in range(H // LANES):
            zbuf[r, pl.ds(q * LANES, LANES)] = jnp.zeros((LANES,), jnp.float32)
        return _
    lax.fori_loop(0, C, _zrow, None)

    def _wipe_acc():
        base = s * ROWS_PER_TILE
        off = 0
        left = ROWS_PER_TILE
        while left > 0:
            n = min(C, left)
            pltpu.sync_copy(zbuf.at[pl.ds(0, n)], acc_sh.at[pl.ds(base + off, n)])
            off += n
            left -= n

    _wipe_acc()
    plsc.subcore_barrier()

    def _vreg_pass(s16, d16, S, q):
        es = plsc.load_gather(eS_v, [s16])
        ed = plsc.load_gather(eD_v, [d16])
        e = _leaky(es + ed)
        m = _leaky(S + ed)
        ex = jnp.exp(e - m)
        ex_v[pl.ds(q * LANES, LANES)] = ex
        plsc.addupdate_scatter(den_v, [d16], ex)

    def _scale_rows(nrows):
        def body(r, _):
            scl = ex_v[r]
            for q in range(H // LANES):
                sl = pl.ds(q * LANES, LANES)
                rows_v[r, sl] = rows_v[r, sl] * scl
            return _
        lax.fori_loop(0, nrows, body, None)

    for k in range(NH):
        pltpu.sync_copy(e_hbm.at[0, k], eS_v)
        pltpu.sync_copy(e_hbm.at[1, k], eD_v)

        def _zden(i, _):
            den_v[pl.ds(i * LANES, LANES)] = jnp.zeros((LANES,), jnp.float32)
            return _
        lax.fori_loop(0, N // LANES, _zden, None)

        def _vmax(i, acc):
            return jnp.maximum(acc, eS_v[pl.ds(i * LANES, LANES)])
        vmax = lax.fori_loop(1, N // LANES, _vmax, eS_v[pl.ds(0, LANES)])
        S = jnp.max(vmax)

        hk = hks[k]

        def _chunk(j, _):
            b = wid * EW + j * C
            pltpu.sync_copy(src_hbm.at[pl.ds(b, C)], src_v)
            pltpu.sync_copy(dst_hbm.at[pl.ds(b, C)], dst_v)
            pltpu.async_copy(hk.at[src_v], rows_v, sem).wait()
            for q in range(C // LANES):
                s16 = src_v[pl.ds(q * LANES, LANES)]
                d16 = dst_v[pl.ds(q * LANES, LANES)]
                _vreg_pass(s16, d16, S, q)
            _scale_rows(C)
            pltpu.sync_copy(rows_v, acc_sh.at[dst_v], add=True)
            return _
        lax.fori_loop(0, NFULL, _chunk, None)

        b = wid * EW + NFULL * C
        pltpu.sync_copy(src_hbm.at[pl.ds(b, REM)], src16)
        pltpu.sync_copy(dst_hbm.at[pl.ds(b, REM)], dst16)
        pltpu.async_copy(hk.at[src16], rows_v.at[pl.ds(0, REM)], sem).wait()
        _vreg_pass(src16[...], dst16[...], S, 0)
        _scale_rows(REM)
        pltpu.sync_copy(rows_v.at[pl.ds(0, REM)], acc_sh.at[dst16], add=True)

        plsc.subcore_barrier()
        pltpu.sync_copy(den_v, den_out.at[k, wid])
        base = s * ROWS_PER_TILE
        pltpu.sync_copy(acc_sh.at[pl.ds(base, ROWS_PER_TILE)],
                        acc_out.at[c, k, pl.ds(base, ROWS_PER_TILE)])
        if k < NH - 1:
            _wipe_acc()
            plsc.subcore_barrier()


def _phase2(h0, h1, h2, h3, src, dst, e_arr):
    mesh = plsc.VectorSubcoreMesh(core_axis_name="c", subcore_axis_name="s",
                                  num_cores=NC, num_subcores=NS)
    return pl.kernel(
        _sc_body,
        out_type=[
            jax.ShapeDtypeStruct((NC, NH, N, H), jnp.float32),
            jax.ShapeDtypeStruct((NH, NW, N), jnp.float32),
        ],
        mesh=mesh,
        scratch_types=[
            pltpu.VMEM((N,), jnp.float32),
            pltpu.VMEM((N,), jnp.float32),
            pltpu.VMEM((N,), jnp.float32),
            pltpu.VMEM((C,), jnp.int32),
            pltpu.VMEM((C,), jnp.int32),
            pltpu.VMEM((C, H), jnp.float32),
            pltpu.VMEM((C,), jnp.float32),
            pltpu.VMEM((C, H), jnp.float32),
            pltpu.VMEM((REM,), jnp.int32),
            pltpu.VMEM((REM,), jnp.int32),
            pltpu.VMEM_SHARED((N, H), jnp.float32),
            pltpu.SemaphoreType.DMA,
        ],
    )(h0, h1, h2, h3, src, dst, e_arr)


R3 = 1000


def _p3_body(acc_ref, den_ref, x_ref, wh_ref, bh_ref, wl_ref, out_ref):
    a = acc_ref[...]
    asum = a[0] + a[1]
    den = jnp.sum(den_ref[...], axis=1)
    parts = []
    for k in range(NH):
        g = asum[k] / (den[k][:, None] + 1e-16)
        parts.append(jnp.where(g > 0, g, jnp.exp(jnp.minimum(g, 0.0)) - 1.0))
    hcat = jnp.concatenate(parts, axis=1)
    z = jnp.dot(hcat, wh_ref[...], preferred_element_type=jnp.float32)
    z = z + bh_ref[...]
    z = jnp.dot(z, wl_ref[...], preferred_element_type=jnp.float32)
    out_ref[...] = z + x_ref[...]


def _phase3(acc, den, x, w_hidden, b_hidden, w_lin):
    return pl.pallas_call(
        _p3_body,
        grid=(N // R3,),
        in_specs=[
            pl.BlockSpec((NC, NH, R3, H), lambda i: (0, 0, i, 0)),
            pl.BlockSpec((NH, NW, R3), lambda i: (0, 0, i)),
            pl.BlockSpec((R3, D), lambda i: (i, 0)),
            pl.BlockSpec((NH * H, LIN), lambda i: (0, 0)),
            pl.BlockSpec((1, LIN), lambda i: (0, 0)),
            pl.BlockSpec((LIN, OUT), lambda i: (0, 0)),
        ],
        out_specs=pl.BlockSpec((R3, OUT), lambda i: (i, 0)),
        out_shape=jax.ShapeDtypeStruct((N, OUT), jnp.float32),
    )(acc, den, x, w_hidden, b_hidden, w_lin)


def kernel(x, edge_index, W, a, W_hidden, b_hidden, W_lin):
    w_cat = jnp.transpose(W, (1, 0, 2)).reshape(D, NH * H)
    a2 = a.reshape(NH, 2, H)
    src = edge_index[0]
    dst = edge_index[1]
    h0, h1, h2, h3, e_arr = _phase1(x, w_cat, a2)
    acc, den = _phase2(h0, h1, h2, h3, src, dst, e_arr)
    return _phase3(acc, den, x, W_hidden, b_hidden.reshape(1, LIN), W_lin)


# speedup vs baseline: 24.3676x; 24.3676x over previous
import functools

import jax
import jax.numpy as jnp
from jax import lax
from jax.experimental import pallas as pl
from jax.experimental.pallas import tpu as pltpu
from jax.experimental.pallas import tpu_sc as plsc

N = 10000
E = 320000
D = 128
H = 64
NH = 4
LIN = 128
OUT = 128
ALPHA = 0.2

NC = 2
NS = 16
LANES = 16
NW = NC * NS
EW = E // NW
C = 128
NFULL = EW // C
REM = EW - NFULL * C
NPAD = 10240
ROWS_PER_TILE = NPAD // NS


def _leaky(u):
    return jnp.where(u > 0, u, ALPHA * u)


R1 = 1024


def _p1_body(x_ref, wc_ref, a2_ref, h0, h1, h2, h3, e_ref):
    xb = x_ref[...]
    houts = (h0, h1, h2, h3)
    for k in range(NH):
        hk = jnp.dot(xb, wc_ref[:, k * H:(k + 1) * H],
                     preferred_element_type=jnp.float32)
        houts[k][...] = hk
        ek = lax.dot_general(a2_ref[k], hk, (((1,), (1,)), ((), ())),
                             preferred_element_type=jnp.float32)
        e_ref[:, k, :] = ek


def _phase1(x, w_cat, a2):
    return pl.pallas_call(
        _p1_body,
        grid=(pl.cdiv(N, R1),),
        in_specs=[
            pl.BlockSpec((R1, D), lambda i: (i, 0)),
            pl.BlockSpec((D, NH * H), lambda i: (0, 0)),
            pl.BlockSpec((NH, 2, H), lambda i: (0, 0, 0)),
        ],
        out_specs=[
            pl.BlockSpec((R1, H), lambda i: (i, 0)),
            pl.BlockSpec((R1, H), lambda i: (i, 0)),
            pl.BlockSpec((R1, H), lambda i: (i, 0)),
            pl.BlockSpec((R1, H), lambda i: (i, 0)),
            pl.BlockSpec((2, NH, R1), lambda i: (0, 0, i)),
        ],
        out_shape=[
            jax.ShapeDtypeStruct((N, H), jnp.float32),
            jax.ShapeDtypeStruct((N, H), jnp.float32),
            jax.ShapeDtypeStruct((N, H), jnp.float32),
            jax.ShapeDtypeStruct((N, H), jnp.float32),
            jax.ShapeDtypeStruct((2, NH, NPAD), jnp.float32),
        ],
    )(x, w_cat, a2)


def _sc_body(h0, h1, h2, h3, src_hbm, dst_hbm, e_hbm,
             acc_out, den_out,
             eS_v, eD_v, den_v, src_v, dst_v, rows_v, ex_v, zbuf,
             src16, dst16, acc_sh, sem):
    c = lax.axis_index("c")
    s = lax.axis_index("s")
    wid = s * NC + c
    hks = (h0, h1, h2, h3)

    def _zrow(r, _):
        for q in range(H // LANES):
            zbuf[r, pl.ds(q * LANES, LANES)] = jnp.zeros((LANES,), jnp.float32)
        return _
    lax.fori_loop(0, C, _zrow, None)

    def _wipe_acc():
        base = s * ROWS_PER_TILE
        off = 0
        left = ROWS_PER_TILE
        while left > 0:
            n = min(C, left)
            pltpu.sync_copy(zbuf.at[pl.ds(0, n)], acc_sh.at[pl.ds(base + off, n)])
            off += n
            left -= n

    _wipe_acc()
    plsc.subcore_barrier()

    def _vreg_pass(s16, d16, S, q):
        es = plsc.load_gather(eS_v, [s16])
        ed = plsc.load_gather(eD_v, [d16])
        e = _leaky(es + ed)
        m = _leaky(S + ed)
        ex = jnp.exp(e - m)
        ex_v[pl.ds(q * LANES, LANES)] = ex
        plsc.addupdate_scatter(den_v, [d16], ex)

    def _scale_rows(nrows):
        def body(g, _):
            exv = ex_v[pl.ds(g * LANES, LANES)]
            for r in range(LANES):
                row = g * LANES + r
                scl = exv[r]
                for q in range(H // LANES):
                    sl = pl.ds(q * LANES, LANES)
                    rows_v[row, sl] = rows_v[row, sl] * scl
            return _
        lax.fori_loop(0, nrows // LANES, body, None)

    for k in range(NH):
        pltpu.sync_copy(e_hbm.at[pl.ds(k * NPAD, N)], eS_v)
        pltpu.sync_copy(e_hbm.at[pl.ds((NH + k) * NPAD, N)], eD_v)

        def _zden(i, _):
            den_v[pl.ds(i * LANES, LANES)] = jnp.zeros((LANES,), jnp.float32)
            return _
        lax.fori_loop(0, N // LANES, _zden, None)

        def _vmax(i, acc):
            return jnp.maximum(acc, eS_v[pl.ds(i * LANES, LANES)])
        vmax = lax.fori_loop(1, N // LANES, _vmax, eS_v[pl.ds(0, LANES)])
        S = jnp.max(vmax)

        hk = hks[k]

        def _chunk(j, _):
            b = wid * EW + j * C
            pltpu.sync_copy(src_hbm.at[pl.ds(b, C)], src_v)
            pltpu.sync_copy(dst_hbm.at[pl.ds(b, C)], dst_v)
            pltpu.async_copy(hk.at[src_v], rows_v, sem).wait()
            for q in range(C // LANES):
                s16 = src_v[pl.ds(q * LANES, LANES)]
                d16 = dst_v[pl.ds(q * LANES, LANES)]
                _vreg_pass(s16, d16, S, q)
            _scale_rows(C)
            pltpu.sync_copy(rows_v, acc_sh.at[dst_v], add=True)
            return _
        lax.fori_loop(0, NFULL, _chunk, None)

        b = wid * EW + NFULL * C
        pltpu.sync_copy(src_hbm.at[pl.ds(b, REM)], src16)
        pltpu.sync_copy(dst_hbm.at[pl.ds(b, REM)], dst16)
        pltpu.async_copy(hk.at[src16], rows_v.at[pl.ds(0, REM)], sem).wait()
        _vreg_pass(src16[...], dst16[...], S, 0)
        _scale_rows(REM)
        pltpu.sync_copy(rows_v.at[pl.ds(0, REM)], acc_sh.at[dst16], add=True)

        plsc.subcore_barrier()
        pltpu.sync_copy(den_v, den_out.at[pl.ds((k * NW + wid) * N, N)])
        base = s * ROWS_PER_TILE
        pltpu.sync_copy(acc_sh.at[pl.ds(base, ROWS_PER_TILE)],
                        acc_out.at[(c * NH + k) * NS + s])
        if k < NH - 1:
            _wipe_acc()
            plsc.subcore_barrier()


def _phase2(h0, h1, h2, h3, src, dst, e_arr):
    mesh = plsc.VectorSubcoreMesh(core_axis_name="c", subcore_axis_name="s",
                                  num_cores=NC, num_subcores=NS)
    return pl.kernel(
        _sc_body,
        out_type=[
            jax.ShapeDtypeStruct((NC * NH * NS, ROWS_PER_TILE, H), jnp.float32),
            jax.ShapeDtypeStruct((NH * NW * N,), jnp.float32),
        ],
        mesh=mesh,
        compiler_params=pltpu.CompilerParams(needs_layout_passes=False,
                                             use_tc_tiling_on_sc=False),
        scratch_types=[
            pltpu.VMEM((N,), jnp.float32),
            pltpu.VMEM((N,), jnp.float32),
            pltpu.VMEM((N,), jnp.float32),
            pltpu.VMEM((C,), jnp.int32),
            pltpu.VMEM((C,), jnp.int32),
            pltpu.VMEM((C, H), jnp.float32),
            pltpu.VMEM((C,), jnp.float32),
            pltpu.VMEM((C, H), jnp.float32),
            pltpu.VMEM((REM,), jnp.int32),
            pltpu.VMEM((REM,), jnp.int32),
            pltpu.VMEM_SHARED((NPAD, H), jnp.float32),
            pltpu.SemaphoreType.DMA,
        ],
    )(h0, h1, h2, h3, src, dst, e_arr)


R3 = 1000


def _p3_body(acc_ref, den_ref, x_ref, wh_ref, bh_ref, wl_ref, out_ref):
    a = acc_ref[...]
    asum = a[0] + a[1]
    den = jnp.sum(den_ref[...], axis=2)
    parts = []
    for k in range(NH):
        g = asum[k] / (den[k][:, None] + 1e-16)
        parts.append(jnp.where(g > 0, g, jnp.exp(jnp.minimum(g, 0.0)) - 1.0))
    hcat = jnp.concatenate(parts, axis=1)
    z = jnp.dot(hcat, wh_ref[...], preferred_element_type=jnp.float32)
    z = z + bh_ref[...]
    z = jnp.dot(z, wl_ref[...], preferred_element_type=jnp.float32)
    out_ref[...] = z + x_ref[...]


def _phase3(acc, den, x, w_hidden, b_hidden, w_lin):
    return pl.pallas_call(
        _p3_body,
        grid=(N // R3,),
        in_specs=[
            pl.BlockSpec((NC, NH, R3, H), lambda i: (0, 0, i, 0)),
            pl.BlockSpec((NH, R3, NW), lambda i: (0, i, 0)),
            pl.BlockSpec((R3, D), lambda i: (i, 0)),
            pl.BlockSpec((NH * H, LIN), lambda i: (0, 0)),
            pl.BlockSpec((1, LIN), lambda i: (0, 0)),
            pl.BlockSpec((LIN, OUT), lambda i: (0, 0)),
        ],
        out_specs=pl.BlockSpec((R3, OUT), lambda i: (i, 0)),
        out_shape=jax.ShapeDtypeStruct((N, OUT), jnp.float32),
    )(acc, den, x, w_hidden, b_hidden, w_lin)


def kernel(x, edge_index, W, a, W_hidden, b_hidden, W_lin):
    w_cat = jnp.transpose(W, (1, 0, 2)).reshape(D, NH * H)
    a2 = a.reshape(NH, 2, H)
    src = edge_index[0]
    dst = edge_index[1]
    h0, h1, h2, h3, e_arr = _phase1(x, w_cat, a2)
    acc_f, den_f = _phase2(h0, h1, h2, h3, src, dst, e_arr.reshape(-1))
    acc = acc_f.reshape(NC, NH, NPAD, H)
    den_t = jnp.transpose(den_f.reshape(NH, NW, N), (0, 2, 1))
    return _phase3(acc, den_t, x, W_hidden, b_hidden.reshape(1, LIN), W_lin)


# speedup vs baseline: 37.4191x; 1.5356x over previous
import jax
import jax.numpy as jnp
from jax import lax
from jax.experimental import pallas as pl
from jax.experimental.pallas import tpu as pltpu
from jax.experimental.pallas import tpu_sc as plsc

N = 10000
E = 320000
D = 128
H = 64
NH = 4
LIN = 128
OUT = 128
ALPHA = 0.2

NC = 2
NS = 16
LANES = 16
NW = NC * NS
EW = E // NW
CH = 80
NCH = EW // CH
NPAIR = (NCH - 1) // 2
NPAD = 10240
RPT = NPAD // NS
ZR = 64


def _leaky(u):
    return jnp.where(u > 0, u, ALPHA * u)


R1 = 1024


def _p1_body(x_ref, wc_ref, a2_ref, h0, h1, h2, h3, e_ref):
    xb = x_ref[...]
    houts = (h0, h1, h2, h3)
    for k in range(NH):
        hk = jnp.dot(xb, wc_ref[:, k * H:(k + 1) * H],
                     preferred_element_type=jnp.float32)
        houts[k][...] = hk
        ek = lax.dot_general(a2_ref[k], hk, (((1,), (1,)), ((), ())),
                             preferred_element_type=jnp.float32)
        e_ref[:, k, :] = ek


def _phase1(x, w_cat, a2):
    return pl.pallas_call(
        _p1_body,
        grid=(pl.cdiv(N, R1),),
        in_specs=[
            pl.BlockSpec((R1, D), lambda i: (i, 0)),
            pl.BlockSpec((D, NH * H), lambda i: (0, 0)),
            pl.BlockSpec((NH, 2, H), lambda i: (0, 0, 0)),
        ],
        out_specs=[
            pl.BlockSpec((R1, H), lambda i: (i, 0)),
            pl.BlockSpec((R1, H), lambda i: (i, 0)),
            pl.BlockSpec((R1, H), lambda i: (i, 0)),
            pl.BlockSpec((R1, H), lambda i: (i, 0)),
            pl.BlockSpec((2, NH, R1), lambda i: (0, 0, i)),
        ],
        out_shape=[
            jax.ShapeDtypeStruct((N, H), jnp.float32),
            jax.ShapeDtypeStruct((N, H), jnp.float32),
            jax.ShapeDtypeStruct((N, H), jnp.float32),
            jax.ShapeDtypeStruct((N, H), jnp.float32),
            jax.ShapeDtypeStruct((2, NH, NPAD), jnp.float32),
        ],
    )(x, w_cat, a2)


def _sc_body(h0, h1, h2, h3, src2_hbm, dst2_hbm, e_hbm,
             acc_out, den_out,
             src_t, dst_t, eS_v, eD_v, den_v,
             rows0, rows1, ex_v, zbuf, acc_sh,
             gsem0, gsem1):
    c = lax.axis_index("c")
    s = lax.axis_index("s")
    wid = s * NC + c
    hks = (h0, h1, h2, h3)

    pltpu.sync_copy(src2_hbm.at[pl.ds(wid * NCH, NCH)], src_t)
    pltpu.sync_copy(dst2_hbm.at[pl.ds(wid * NCH, NCH)], dst_t)

    def _zrow(r, carry):
        for q in range(H // LANES):
            zbuf[r, pl.ds(q * LANES, LANES)] = jnp.zeros((LANES,), jnp.float32)
        return carry
    lax.fori_loop(0, ZR, _zrow, None)

    def _wipe_acc():
        base = s * RPT
        for i in range(RPT // ZR):
            pltpu.sync_copy(zbuf, acc_sh.at[pl.ds(base + i * ZR, ZR)])

    _wipe_acc()
    plsc.subcore_barrier()

    for k in range(NH):
        hk = hks[k]
        pltpu.sync_copy(e_hbm.at[pl.ds(k * NPAD, N)], eS_v)
        pltpu.sync_copy(e_hbm.at[pl.ds((NH + k) * NPAD, N)], eD_v)

        def _zden(i, carry):
            den_v[pl.ds(i * LANES, LANES)] = jnp.zeros((LANES,), jnp.float32)
            return carry
        lax.fori_loop(0, N // LANES, _zden, None)

        def _vmax(i, acc):
            return jnp.maximum(acc, eS_v[pl.ds(i * LANES, LANES)])
        vmax = lax.fori_loop(1, N // LANES, _vmax, eS_v[pl.ds(0, LANES)])
        S = plsc.cummax(vmax)[LANES - 1]

        def _fire(jj, rbuf, sem):
            pltpu.async_copy(hk.at[src_t.at[jj]], rbuf, sem)

        def _wait(jj, rbuf, sem):
            pltpu.make_async_copy(hk.at[src_t.at[jj]], rbuf, sem).wait()

        def _process(j, rbuf):
            for q in range(CH // LANES):
                sl = pl.ds(q * LANES, LANES)
                s16 = src_t[j, sl]
                d16 = dst_t[j, sl]
                ed = plsc.load_gather(eD_v, [d16])
                ex = jnp.exp(_leaky(plsc.load_gather(eS_v, [s16]) + ed)
                             - _leaky(S + ed))
                ex_v[sl] = ex
                plsc.addupdate_scatter(den_v, [d16], ex)

            def scale(g, carry):
                ev = ex_v[pl.ds(g * LANES, LANES)]
                for r in range(LANES):
                    row = g * LANES + r
                    scl = ev[r]
                    for q in range(H // LANES):
                        sl0 = pl.ds(q * LANES, LANES)
                        rbuf[row, sl0] = rbuf[row, sl0] * scl
                return carry
            lax.fori_loop(0, CH // LANES, scale, None)
            pltpu.sync_copy(rbuf, acc_sh.at[dst_t.at[j]], add=True)

        _fire(0, rows0, gsem0)

        def _pair(i, carry):
            j0 = 2 * i
            j1 = j0 + 1
            _fire(j1, rows1, gsem1)
            _wait(j0, rows0, gsem0)
            _process(j0, rows0)
            _fire(j0 + 2, rows0, gsem0)
            _wait(j1, rows1, gsem1)
            _process(j1, rows1)
            return carry
        lax.fori_loop(0, NPAIR, _pair, None)
        _wait(NCH - 1, rows0, gsem0)
        _process(NCH - 1, rows0)

        plsc.subcore_barrier()
        pltpu.sync_copy(den_v, den_out.at[pl.ds((k * NW + wid) * N, N)])
        pltpu.sync_copy(acc_sh.at[pl.ds(s * RPT, RPT)],
                        acc_out.at[(c * NH + k) * NS + s])
        if k < NH - 1:
            _wipe_acc()
            plsc.subcore_barrier()


def _phase2(h0, h1, h2, h3, src2, dst2, e_flat):
    mesh = plsc.VectorSubcoreMesh(core_axis_name="c", subcore_axis_name="s",
                                  num_cores=NC, num_subcores=NS)
    return pl.kernel(
        _sc_body,
        out_type=[
            jax.ShapeDtypeStruct((NC * NH * NS, RPT, H), jnp.float32),
            jax.ShapeDtypeStruct((NH * NW * N,), jnp.float32),
        ],
        mesh=mesh,
        compiler_params=pltpu.CompilerParams(needs_layout_passes=False,
                                             use_tc_tiling_on_sc=False),
        scratch_types=[
            pltpu.VMEM((NCH, CH), jnp.int32),
            pltpu.VMEM((NCH, CH), jnp.int32),
            pltpu.VMEM((N,), jnp.float32),
            pltpu.VMEM((N,), jnp.float32),
            pltpu.VMEM((N,), jnp.float32),
            pltpu.VMEM((CH, H), jnp.float32),
            pltpu.VMEM((CH, H), jnp.float32),
            pltpu.VMEM((CH,), jnp.float32),
            pltpu.VMEM((ZR, H), jnp.float32),
            pltpu.VMEM_SHARED((NPAD, H), jnp.float32),
            pltpu.SemaphoreType.DMA,
            pltpu.SemaphoreType.DMA,
        ],
    )(h0, h1, h2, h3, src2, dst2, e_flat)


R3 = 1000


def _p3_body(acc_ref, den_ref, x_ref, wh_ref, bh_ref, wl_ref, out_ref):
    a = acc_ref[...]
    asum = a[0] + a[1]
    den = jnp.sum(den_ref[...], axis=2)
    parts = []
    for k in range(NH):
        g = asum[k] / (den[k][:, None] + 1e-16)
        parts.append(jnp.where(g > 0, g, jnp.exp(jnp.minimum(g, 0.0)) - 1.0))
    hcat = jnp.concatenate(parts, axis=1)
    z = jnp.dot(hcat, wh_ref[...], preferred_element_type=jnp.float32)
    z = z + bh_ref[...]
    z = jnp.dot(z, wl_ref[...], preferred_element_type=jnp.float32)
    out_ref[...] = z + x_ref[...]


def _phase3(acc, den, x, w_hidden, b_hidden, w_lin):
    return pl.pallas_call(
        _p3_body,
        grid=(N // R3,),
        in_specs=[
            pl.BlockSpec((NC, NH, R3, H), lambda i: (0, 0, i, 0)),
            pl.BlockSpec((NH, R3, NW), lambda i: (0, i, 0)),
            pl.BlockSpec((R3, D), lambda i: (i, 0)),
            pl.BlockSpec((NH * H, LIN), lambda i: (0, 0)),
            pl.BlockSpec((1, LIN), lambda i: (0, 0)),
            pl.BlockSpec((LIN, OUT), lambda i: (0, 0)),
        ],
        out_specs=pl.BlockSpec((R3, OUT), lambda i: (i, 0)),
        out_shape=jax.ShapeDtypeStruct((N, OUT), jnp.float32),
    )(acc, den, x, w_hidden, b_hidden, w_lin)


def kernel(x, edge_index, W, a, W_hidden, b_hidden, W_lin):
    w_cat = jnp.transpose(W, (1, 0, 2)).reshape(D, NH * H)
    a2 = a.reshape(NH, 2, H)
    src2 = edge_index[0].reshape(E // CH, CH)
    dst2 = edge_index[1].reshape(E // CH, CH)
    h0, h1, h2, h3, e_arr = _phase1(x, w_cat, a2)
    acc_f, den_f = _phase2(h0, h1, h2, h3, src2, dst2, e_arr.reshape(-1))
    acc = acc_f.reshape(NC, NH, NPAD, H)
    den_t = jnp.transpose(den_f.reshape(NH, NW, N), (0, 2, 1))
    return _phase3(acc, den_t, x, W_hidden, b_hidden.reshape(1, LIN), W_lin)


# speedup vs baseline: 42.2749x; 1.1298x over previous
import jax
import jax.numpy as jnp
from jax import lax
from jax.experimental import pallas as pl
from jax.experimental.pallas import tpu as pltpu
from jax.experimental.pallas import tpu_sc as plsc

N = 10000
E = 320000
D = 128
H = 64
NH = 4
LIN = 128
OUT = 128
ALPHA = 0.2

NC = 2
NS = 16
LANES = 16
NW = NC * NS
EW = E // NW
CH = 80
NCH = EW // CH
NPAIR = (NCH - 1) // 2
NPAD = 10240
RPT = NPAD // NS
ZR = 64


def _leaky(u):
    return jnp.where(u > 0, u, ALPHA * u)


R1 = 1024


def _p1_body(x_ref, wc_ref, a2_ref, h0, h1, h2, h3, e_ref):
    xb = x_ref[...]
    houts = (h0, h1, h2, h3)
    for k in range(NH):
        hk = jnp.dot(xb, wc_ref[:, k * H:(k + 1) * H],
                     preferred_element_type=jnp.float32)
        houts[k][...] = hk
        ek = lax.dot_general(a2_ref[k], hk, (((1,), (1,)), ((), ())),
                             preferred_element_type=jnp.float32)
        e_ref[:, k, :] = ek


def _phase1(x, w_cat, a2):
    return pl.pallas_call(
        _p1_body,
        grid=(pl.cdiv(N, R1),),
        in_specs=[
            pl.BlockSpec((R1, D), lambda i: (i, 0)),
            pl.BlockSpec((D, NH * H), lambda i: (0, 0)),
            pl.BlockSpec((NH, 2, H), lambda i: (0, 0, 0)),
        ],
        out_specs=[
            pl.BlockSpec((R1, H), lambda i: (i, 0)),
            pl.BlockSpec((R1, H), lambda i: (i, 0)),
            pl.BlockSpec((R1, H), lambda i: (i, 0)),
            pl.BlockSpec((R1, H), lambda i: (i, 0)),
            pl.BlockSpec((2, NH, R1), lambda i: (0, 0, i)),
        ],
        out_shape=[
            jax.ShapeDtypeStruct((N, H), jnp.float32),
            jax.ShapeDtypeStruct((N, H), jnp.float32),
            jax.ShapeDtypeStruct((N, H), jnp.float32),
            jax.ShapeDtypeStruct((N, H), jnp.float32),
            jax.ShapeDtypeStruct((2, NH, NPAD), jnp.float32),
        ],
    )(x, w_cat, a2)


def _sc_body(h0, h1, h2, h3, src2_hbm, dst2_hbm, e_hbm,
             acc_out, den_out,
             src_t, dst_t, eS_v, eD_v, den_v,
             rows0, rows1, rows2, ex_v, zbuf, acc_sh,
             gsem0, gsem1, gsem2, ssem0, ssem1, ssem2):
    c = lax.axis_index("c")
    s = lax.axis_index("s")
    wid = s * NC + c
    hks = (h0, h1, h2, h3)

    pltpu.sync_copy(src2_hbm.at[pl.ds(wid * NCH, NCH)], src_t)
    pltpu.sync_copy(dst2_hbm.at[pl.ds(wid * NCH, NCH)], dst_t)

    def _zrow(r, carry):
        for q in range(H // LANES):
            zbuf[r, pl.ds(q * LANES, LANES)] = jnp.zeros((LANES,), jnp.float32)
        return carry
    lax.fori_loop(0, ZR, _zrow, None)

    def _wipe_acc():
        base = s * RPT
        for i in range(RPT // ZR):
            pltpu.sync_copy(zbuf, acc_sh.at[pl.ds(base + i * ZR, ZR)])

    _wipe_acc()
    plsc.subcore_barrier()

    for k in range(NH):
        hk = hks[k]
        pltpu.sync_copy(e_hbm.at[pl.ds(k * NPAD, N)], eS_v)
        pltpu.sync_copy(e_hbm.at[pl.ds((NH + k) * NPAD, N)], eD_v)

        def _zden(i, carry):
            den_v[pl.ds(i * LANES, LANES)] = jnp.zeros((LANES,), jnp.float32)
            return carry
        lax.fori_loop(0, N // LANES, _zden, None)

        def _vmax(i, acc):
            return jnp.maximum(acc, eS_v[pl.ds(i * LANES, LANES)])
        vmax = lax.fori_loop(1, N // LANES, _vmax, eS_v[pl.ds(0, LANES)])
        S = plsc.cummax(vmax)[LANES - 1]

        bufs = (rows0, rows1, rows2)
        gsems = (gsem0, gsem1, gsem2)
        ssems = (ssem0, ssem1, ssem2)

        def _fire(jj, rbuf, sem):
            pltpu.async_copy(hk.at[src_t.at[jj]], rbuf, sem)

        def _wait_g(jj, rbuf, sem):
            pltpu.make_async_copy(hk.at[src_t.at[jj]], rbuf, sem).wait()

        def _fire_s(jj, rbuf, sem):
            pltpu.async_copy(rbuf, acc_sh.at[dst_t.at[jj]], sem, add=True)

        def _wait_s(rbuf, sem):
            pltpu.make_async_copy(rbuf, acc_sh.at[dst_t.at[0]], sem).wait()

        def _compute(j, rbuf):
            def expass(q, carry):
                sl = pl.ds(q * LANES, LANES)
                s16 = src_t[j, sl]
                d16 = dst_t[j, sl]
                ed = plsc.load_gather(eD_v, [d16])
                ex = jnp.exp(_leaky(plsc.load_gather(eS_v, [s16]) + ed)
                             - _leaky(S + ed))
                ex_v[sl] = ex
                plsc.addupdate_scatter(den_v, [d16], ex)
                return carry
            lax.fori_loop(0, CH // LANES, expass, None)

            def scale(g, carry):
                ev = ex_v[pl.ds(g * LANES, LANES)]
                for r in range(LANES):
                    row = g * LANES + r
                    scl = ev[r]
                    for q in range(H // LANES):
                        sl0 = pl.ds(q * LANES, LANES)
                        rbuf[row, sl0] = rbuf[row, sl0] * scl
                return carry
            lax.fori_loop(0, CH // LANES, scale, None)

        NTRI = (NCH - 2) // 3
        _fire(0, rows0, gsem0)

        def _triple(i, carry):
            for t in range(3):
                j = 3 * i + t
                cur, nxt = bufs[t], bufs[(t + 1) % 3]
                _wait_g(j, cur, gsems[t])
                if t < 2:
                    @pl.when(i > 0)
                    def _():
                        _wait_s(nxt, ssems[(t + 1) % 3])
                else:
                    _wait_s(nxt, ssems[0])
                _fire(j + 1, nxt, gsems[(t + 1) % 3])
                _compute(j, cur)
                _fire_s(j, cur, ssems[t])
            return carry
        lax.fori_loop(0, NTRI, _triple, None)
        jt = 3 * NTRI
        _wait_g(jt, rows0, gsem0)
        _wait_s(rows1, ssem1)
        _fire(jt + 1, rows1, gsem1)
        _compute(jt, rows0)
        _fire_s(jt, rows0, ssem0)
        _wait_g(jt + 1, rows1, gsem1)
        _compute(jt + 1, rows1)
        _fire_s(jt + 1, rows1, ssem1)
        _wait_s(rows0, ssem0)
        _wait_s(rows1, ssem1)
        _wait_s(rows2, ssem2)

        plsc.subcore_barrier()
        pltpu.sync_copy(den_v, den_out.at[pl.ds((k * NW + wid) * N, N)])
        pltpu.sync_copy(acc_sh.at[pl.ds(s * RPT, RPT)],
                        acc_out.at[(c * NH + k) * NS + s])
        if k < NH - 1:
            _wipe_acc()
            plsc.subcore_barrier()


def _phase2(h0, h1, h2, h3, src2, dst2, e_flat):
    mesh = plsc.VectorSubcoreMesh(core_axis_name="c", subcore_axis_name="s",
                                  num_cores=NC, num_subcores=NS)
    return pl.kernel(
        _sc_body,
        out_type=[
            jax.ShapeDtypeStruct((NC * NH * NS, RPT, H), jnp.float32),
            jax.ShapeDtypeStruct((NH * NW * N,), jnp.float32),
        ],
        mesh=mesh,
        compiler_params=pltpu.CompilerParams(needs_layout_passes=False,
                                             use_tc_tiling_on_sc=False),
        scratch_types=[
            pltpu.VMEM((NCH, CH), jnp.int32),
            pltpu.VMEM((NCH, CH), jnp.int32),
            pltpu.VMEM((N,), jnp.float32),
            pltpu.VMEM((N,), jnp.float32),
            pltpu.VMEM((N,), jnp.float32),
            pltpu.VMEM((CH, H), jnp.float32),
            pltpu.VMEM((CH, H), jnp.float32),
            pltpu.VMEM((CH, H), jnp.float32),
            pltpu.VMEM((CH,), jnp.float32),
            pltpu.VMEM((ZR, H), jnp.float32),
            pltpu.VMEM_SHARED((NPAD, H), jnp.float32),
            pltpu.SemaphoreType.DMA,
            pltpu.SemaphoreType.DMA,
            pltpu.SemaphoreType.DMA,
            pltpu.SemaphoreType.DMA,
            pltpu.SemaphoreType.DMA,
            pltpu.SemaphoreType.DMA,
        ],
    )(h0, h1, h2, h3, src2, dst2, e_flat)


R3 = 1000


def _p3_body(acc_ref, den_ref, x_ref, wh_ref, bh_ref, wl_ref, out_ref):
    a = acc_ref[...]
    asum = a[0] + a[1]
    den = jnp.sum(den_ref[...], axis=2)
    parts = []
    for k in range(NH):
        g = asum[k] / (den[k][:, None] + 1e-16)
        parts.append(jnp.where(g > 0, g, jnp.exp(jnp.minimum(g, 0.0)) - 1.0))
    hcat = jnp.concatenate(parts, axis=1)
    z = jnp.dot(hcat, wh_ref[...], preferred_element_type=jnp.float32)
    z = z + bh_ref[...]
    z = jnp.dot(z, wl_ref[...], preferred_element_type=jnp.float32)
    out_ref[...] = z + x_ref[...]


def _phase3(acc, den, x, w_hidden, b_hidden, w_lin):
    return pl.pallas_call(
        _p3_body,
        grid=(N // R3,),
        in_specs=[
            pl.BlockSpec((NC, NH, R3, H), lambda i: (0, 0, i, 0)),
            pl.BlockSpec((NH, R3, NW), lambda i: (0, i, 0)),
            pl.BlockSpec((R3, D), lambda i: (i, 0)),
            pl.BlockSpec((NH * H, LIN), lambda i: (0, 0)),
            pl.BlockSpec((1, LIN), lambda i: (0, 0)),
            pl.BlockSpec((LIN, OUT), lambda i: (0, 0)),
        ],
        out_specs=pl.BlockSpec((R3, OUT), lambda i: (i, 0)),
        out_shape=jax.ShapeDtypeStruct((N, OUT), jnp.float32),
    )(acc, den, x, w_hidden, b_hidden, w_lin)


def kernel(x, edge_index, W, a, W_hidden, b_hidden, W_lin):
    w_cat = jnp.transpose(W, (1, 0, 2)).reshape(D, NH * H)
    a2 = a.reshape(NH, 2, H)
    src2 = edge_index[0].reshape(E // CH, CH)
    dst2 = edge_index[1].reshape(E // CH, CH)
    h0, h1, h2, h3, e_arr = _phase1(x, w_cat, a2)
    acc_f, den_f = _phase2(h0, h1, h2, h3, src2, dst2, e_arr.reshape(-1))
    acc = acc_f.reshape(NC, NH, NPAD, H)
    den_t = jnp.transpose(den_f.reshape(NH, NW, N), (0, 2, 1))
    return _phase3(acc, den_t, x, W_hidden, b_hidden.reshape(1, LIN), W_lin)


# speedup vs baseline: 61.7748x; 1.4613x over previous
import jax
import jax.numpy as jnp
from jax import lax
from jax.experimental import pallas as pl
from jax.experimental.pallas import tpu as pltpu
from jax.experimental.pallas import tpu_sc as plsc

N = 10000
E = 320000
D = 128
H = 64
NH = 4
LIN = 128
OUT = 128
ALPHA = 0.2

NC = 2
NS = 16
LANES = 16
NW = NC * NS
EW = E // NW
CH = 80
NCH = EW // CH
NPAIR = (NCH - 1) // 2
NPAD = 10240
RPT = NPAD // NS
ZR = 64


def _leaky(u):
    return jnp.where(u > 0, u, ALPHA * u)


R1 = 1024


def _p1_body(x_ref, wc_ref, a2_ref, h0, h1, h2, h3, e_ref):
    xb = x_ref[...]
    houts = (h0, h1, h2, h3)
    for k in range(NH):
        hk = jnp.dot(xb, wc_ref[:, k * H:(k + 1) * H],
                     preferred_element_type=jnp.float32)
        houts[k][...] = hk
        ek = lax.dot_general(a2_ref[k], hk, (((1,), (1,)), ((), ())),
                             preferred_element_type=jnp.float32)
        e_ref[:, k, :] = ek


def _phase1(x, w_cat, a2):
    return pl.pallas_call(
        _p1_body,
        grid=(pl.cdiv(N, R1),),
        in_specs=[
            pl.BlockSpec((R1, D), lambda i: (i, 0)),
            pl.BlockSpec((D, NH * H), lambda i: (0, 0)),
            pl.BlockSpec((NH, 2, H), lambda i: (0, 0, 0)),
        ],
        out_specs=[
            pl.BlockSpec((R1, H), lambda i: (i, 0)),
            pl.BlockSpec((R1, H), lambda i: (i, 0)),
            pl.BlockSpec((R1, H), lambda i: (i, 0)),
            pl.BlockSpec((R1, H), lambda i: (i, 0)),
            pl.BlockSpec((2, NH, R1), lambda i: (0, 0, i)),
        ],
        out_shape=[
            jax.ShapeDtypeStruct((N, H), jnp.float32),
            jax.ShapeDtypeStruct((N, H), jnp.float32),
            jax.ShapeDtypeStruct((N, H), jnp.float32),
            jax.ShapeDtypeStruct((N, H), jnp.float32),
            jax.ShapeDtypeStruct((2, NH, NPAD), jnp.float32),
        ],
    )(x, w_cat, a2)


def _sc_body(h0, h1, h2, h3, src2_hbm, dst2_hbm, e_hbm,
             acc_out, den_out,
             src_t, dst_t, eS_v, eD_v, den_v,
             rows0, rows1, rows2, sb0, sb1, sb2, ex_v, zbuf, acc_sh,
             gsem0, gsem1, gsem2, ssem0, ssem1, ssem2):
    c = lax.axis_index("c")
    s = lax.axis_index("s")
    wid = s * NC + c
    hks = (h0, h1, h2, h3)

    pltpu.sync_copy(src2_hbm.at[pl.ds(wid * NCH, NCH)], src_t)
    pltpu.sync_copy(dst2_hbm.at[pl.ds(wid * NCH, NCH)], dst_t)

    def _zrow(r, carry):
        for q in range(H // LANES):
            zbuf[r, pl.ds(q * LANES, LANES)] = jnp.zeros((LANES,), jnp.float32)
        return carry
    lax.fori_loop(0, ZR, _zrow, None)

    def _wipe_acc():
        base = s * RPT
        for i in range(RPT // ZR):
            pltpu.sync_copy(zbuf, acc_sh.at[pl.ds(base + i * ZR, ZR)])

    _wipe_acc()
    plsc.subcore_barrier()

    for k in range(NH):
        hk = hks[k]
        pltpu.sync_copy(e_hbm.at[pl.ds(k * NPAD, N)], eS_v)
        pltpu.sync_copy(e_hbm.at[pl.ds((NH + k) * NPAD, N)], eD_v)

        def _zden(i, carry):
            den_v[pl.ds(i * LANES, LANES)] = jnp.zeros((LANES,), jnp.float32)
            return carry
        lax.fori_loop(0, N // LANES, _zden, None)

        def _vmax(i, acc):
            return jnp.maximum(acc, eS_v[pl.ds(i * LANES, LANES)])
        vmax = lax.fori_loop(1, N // LANES, _vmax, eS_v[pl.ds(0, LANES)])
        S = plsc.cummax(vmax)[LANES - 1]

        bufs = (rows0, rows1, rows2)
        sbufs = (sb0, sb1, sb2)
        gsems = (gsem0, gsem1, gsem2)
        ssems = (ssem0, ssem1, ssem2)

        def _fire(jj, rbuf, sem):
            pltpu.async_copy(hk.at[src_t.at[jj]], rbuf, sem)

        def _wait_g(jj, rbuf, sem):
            pltpu.make_async_copy(hk.at[src_t.at[jj]], rbuf, sem).wait()

        def _fire_s(jj, sbuf, sem):
            pltpu.async_copy(sbuf, acc_sh.at[dst_t.at[jj]], sem, add=True)

        def _wait_s(sbuf, sem):
            pltpu.make_async_copy(sbuf, acc_sh.at[dst_t.at[0]], sem).wait()

        def _expass(j):
            def body(q, carry):
                sl = pl.ds(q * LANES, LANES)
                s16 = src_t[j, sl]
                d16 = dst_t[j, sl]
                ed = plsc.load_gather(eD_v, [d16])
                ex = jnp.exp(_leaky(plsc.load_gather(eS_v, [s16]) + ed)
                             - _leaky(S + ed))
                ex_v[sl] = ex
                plsc.addupdate_scatter(den_v, [d16], ex)
                return carry
            lax.fori_loop(0, CH // LANES, body, None)

        def _scale(rbuf, sbuf):
            def body(g):
                ev = ex_v[pl.ds(g * LANES, LANES)]
                for r in range(LANES):
                    row = g * LANES + r
                    scl = ev[r]
                    vals = [rbuf[row, pl.ds(q * LANES, LANES)] * scl
                            for q in range(H // LANES)]
                    for q in range(H // LANES):
                        sbuf[row, pl.ds(q * LANES, LANES)] = vals[q]
            plsc.parallel_loop(0, CH // LANES)(body)

        NTRI = (NCH - 2) // 3
        _fire(0, rows0, gsem0)

        def _step(i, t, j):
            cur, nxt = bufs[t], bufs[(t + 1) % 3]
            scur = sbufs[t]
            _wait_g(j, cur, gsems[t])
            _fire(j + 1, nxt, gsems[(t + 1) % 3])
            _expass(j)
            if i is None:
                _wait_s(scur, ssems[t])
            else:
                @pl.when(i > 0)
                def _():
                    _wait_s(scur, ssems[t])
            _scale(cur, scur)
            _fire_s(j, scur, ssems[t])

        def _triple(i, carry):
            for t in range(3):
                _step(i, t, 3 * i + t)
            return carry
        lax.fori_loop(0, NTRI, _triple, None)
        jt = 3 * NTRI
        _step(None, 0, jt)
        _wait_g(jt + 1, rows1, gsem1)
        _expass(jt + 1)
        _wait_s(sb1, ssem1)
        _scale(rows1, sb1)
        _fire_s(jt + 1, sb1, ssem1)
        _wait_s(sb0, ssem0)
        _wait_s(sb1, ssem1)
        _wait_s(sb2, ssem2)

        plsc.subcore_barrier()
        pltpu.sync_copy(den_v, den_out.at[pl.ds((k * NW + wid) * N, N)])
        pltpu.sync_copy(acc_sh.at[pl.ds(s * RPT, RPT)],
                        acc_out.at[(c * NH + k) * NS + s])
        if k < NH - 1:
            _wipe_acc()
            plsc.subcore_barrier()


def _phase2(h0, h1, h2, h3, src2, dst2, e_flat):
    mesh = plsc.VectorSubcoreMesh(core_axis_name="c", subcore_axis_name="s",
                                  num_cores=NC, num_subcores=NS)
    return pl.kernel(
        _sc_body,
        out_type=[
            jax.ShapeDtypeStruct((NC * NH * NS, RPT, H), jnp.float32),
            jax.ShapeDtypeStruct((NH * NW * N,), jnp.float32),
        ],
        mesh=mesh,
        compiler_params=pltpu.CompilerParams(needs_layout_passes=False,
                                             use_tc_tiling_on_sc=False),
        scratch_types=[
            pltpu.VMEM((NCH, CH), jnp.int32),
            pltpu.VMEM((NCH, CH), jnp.int32),
            pltpu.VMEM((N,), jnp.float32),
            pltpu.VMEM((N,), jnp.float32),
            pltpu.VMEM((N,), jnp.float32),
            pltpu.VMEM((CH, H), jnp.float32),
            pltpu.VMEM((CH, H), jnp.float32),
            pltpu.VMEM((CH, H), jnp.float32),
            pltpu.VMEM((CH, H), jnp.float32),
            pltpu.VMEM((CH, H), jnp.float32),
            pltpu.VMEM((CH, H), jnp.float32),
            pltpu.VMEM((CH,), jnp.float32),
            pltpu.VMEM((ZR, H), jnp.float32),
            pltpu.VMEM_SHARED((NPAD, H), jnp.float32),
            pltpu.SemaphoreType.DMA,
            pltpu.SemaphoreType.DMA,
            pltpu.SemaphoreType.DMA,
            pltpu.SemaphoreType.DMA,
            pltpu.SemaphoreType.DMA,
            pltpu.SemaphoreType.DMA,
        ],
    )(h0, h1, h2, h3, src2, dst2, e_flat)


R3 = 1000


def _p3_body(acc_ref, den_ref, x_ref, wh_ref, bh_ref, wl_ref, out_ref):
    a = acc_ref[...]
    asum = a[0] + a[1]
    den = jnp.sum(den_ref[...], axis=2)
    parts = []
    for k in range(NH):
        g = asum[k] / (den[k][:, None] + 1e-16)
        parts.append(jnp.where(g > 0, g, jnp.exp(jnp.minimum(g, 0.0)) - 1.0))
    hcat = jnp.concatenate(parts, axis=1)
    z = jnp.dot(hcat, wh_ref[...], preferred_element_type=jnp.float32)
    z = z + bh_ref[...]
    z = jnp.dot(z, wl_ref[...], preferred_element_type=jnp.float32)
    out_ref[...] = z + x_ref[...]


def _phase3(acc, den, x, w_hidden, b_hidden, w_lin):
    return pl.pallas_call(
        _p3_body,
        grid=(N // R3,),
        in_specs=[
            pl.BlockSpec((NC, NH, R3, H), lambda i: (0, 0, i, 0)),
            pl.BlockSpec((NH, R3, NW), lambda i: (0, i, 0)),
            pl.BlockSpec((R3, D), lambda i: (i, 0)),
            pl.BlockSpec((NH * H, LIN), lambda i: (0, 0)),
            pl.BlockSpec((1, LIN), lambda i: (0, 0)),
            pl.BlockSpec((LIN, OUT), lambda i: (0, 0)),
        ],
        out_specs=pl.BlockSpec((R3, OUT), lambda i: (i, 0)),
        out_shape=jax.ShapeDtypeStruct((N, OUT), jnp.float32),
    )(acc, den, x, w_hidden, b_hidden, w_lin)


def kernel(x, edge_index, W, a, W_hidden, b_hidden, W_lin):
    w_cat = jnp.transpose(W, (1, 0, 2)).reshape(D, NH * H)
    a2 = a.reshape(NH, 2, H)
    src2 = edge_index[0].reshape(E // CH, CH)
    dst2 = edge_index[1].reshape(E // CH, CH)
    h0, h1, h2, h3, e_arr = _phase1(x, w_cat, a2)
    acc_f, den_f = _phase2(h0, h1, h2, h3, src2, dst2, e_arr.reshape(-1))
    acc = acc_f.reshape(NC, NH, NPAD, H)
    den_t = jnp.transpose(den_f.reshape(NH, NW, N), (0, 2, 1))
    return _phase3(acc, den_t, x, W_hidden, b_hidden.reshape(1, LIN), W_lin)


# speedup vs baseline: 62.8577x; 1.0175x over previous
import jax
import jax.numpy as jnp
from jax import lax
from jax.experimental import pallas as pl
from jax.experimental.pallas import tpu as pltpu
from jax.experimental.pallas import tpu_sc as plsc

N = 10000
E = 320000
D = 128
H = 64
NH = 4
LIN = 128
OUT = 128
ALPHA = 0.2

NC = 2
NS = 16
LANES = 16
NW = NC * NS
EW = E // NW
CH = 80
NCH = EW // CH
NPAIR = (NCH - 1) // 2
NPAD = 10240
RPT = NPAD // NS
ZR = 64
HW2 = H + LANES


def _leaky(u):
    return jnp.where(u > 0, u, ALPHA * u)


R1 = 1024


def _p1_body(x_ref, wc_ref, a2_ref, h_ref, e_ref):
    xb = x_ref[...]
    for k in range(NH):
        hk = jnp.dot(xb, wc_ref[:, k * H:(k + 1) * H],
                     preferred_element_type=jnp.float32)
        h_ref[k] = hk
        ek = lax.dot_general(a2_ref[k], hk, (((1,), (1,)), ((), ())),
                             preferred_element_type=jnp.float32)
        e_ref[:, k, :] = ek


def _phase1(x, w_cat, a2):
    return pl.pallas_call(
        _p1_body,
        grid=(pl.cdiv(N, R1),),
        in_specs=[
            pl.BlockSpec((R1, D), lambda i: (i, 0)),
            pl.BlockSpec((D, NH * H), lambda i: (0, 0)),
            pl.BlockSpec((NH, 2, H), lambda i: (0, 0, 0)),
        ],
        out_specs=[
            pl.BlockSpec((NH, R1, H), lambda i: (0, i, 0)),
            pl.BlockSpec((2, NH, R1), lambda i: (0, 0, i)),
        ],
        out_shape=[
            jax.ShapeDtypeStruct((NH, N, H), jnp.float32),
            jax.ShapeDtypeStruct((2, NH, NPAD), jnp.float32),
        ],
    )(x, w_cat, a2)


def _sc_body(h_hbm, src2_hbm, dst2_hbm, e_hbm,
             acc_out,
             src_t, dst_t, eS_v, eD_v,
             rows0, rows1, rows2, sb0, sb1, sb2, ex_v, zbuf, acc_sh,
             gsem0, gsem1, gsem2, ssem0, ssem1, ssem2):
    c = lax.axis_index("c")
    s = lax.axis_index("s")
    wid = s * NC + c

    pltpu.sync_copy(src2_hbm.at[pl.ds(wid * NCH, NCH)], src_t)
    pltpu.sync_copy(dst2_hbm.at[pl.ds(wid * NCH, NCH)], dst_t)

    def _zrow(r, carry):
        for q in range(HW2 // LANES):
            zbuf[r, pl.ds(q * LANES, LANES)] = jnp.zeros((LANES,), jnp.float32)
        return carry
    lax.fori_loop(0, ZR, _zrow, None)

    def _wipe_acc():
        base = s * RPT
        for i in range(RPT // ZR):
            pltpu.sync_copy(zbuf, acc_sh.at[pl.ds(base + i * ZR, ZR)])

    _wipe_acc()

    def _head(k, carry):
        hk = h_hbm.at[k]
        pltpu.sync_copy(e_hbm.at[pl.ds(k * NPAD, N)], eS_v)
        pltpu.sync_copy(e_hbm.at[pl.ds((NH + k) * NPAD, N)], eD_v)

        def _vmax(i, acc):
            return jnp.maximum(acc, eS_v[pl.ds(i * LANES, LANES)])
        vmax = lax.fori_loop(1, N // LANES, _vmax, eS_v[pl.ds(0, LANES)])
        S = plsc.cummax(vmax)[LANES - 1]

        plsc.subcore_barrier()

        bufs = (rows0, rows1, rows2)
        sbufs = (sb0, sb1, sb2)
        gsems = (gsem0, gsem1, gsem2)
        ssems = (ssem0, ssem1, ssem2)

        def _fire(jj, rbuf, sem):
            pltpu.async_copy(hk.at[src_t.at[jj]], rbuf, sem)

        def _wait_g(jj, rbuf, sem):
            pltpu.make_async_copy(hk.at[src_t.at[jj]], rbuf, sem).wait()

        def _fire_s(jj, sbuf, sem):
            pltpu.async_copy(sbuf, acc_sh.at[dst_t.at[jj]], sem, add=True)

        def _wait_s(sbuf, sem):
            pltpu.make_async_copy(sbuf, acc_sh.at[dst_t.at[0]], sem).wait()

        def _expass(j):
            def body(q, carry):
                sl = pl.ds(q * LANES, LANES)
                s16 = src_t[j, sl]
                d16 = dst_t[j, sl]
                ed = plsc.load_gather(eD_v, [d16])
                ex = jnp.exp(_leaky(plsc.load_gather(eS_v, [s16]) + ed)
                             - _leaky(S + ed))
                ex_v[sl] = ex
                return carry
            lax.fori_loop(0, CH // LANES, body, None)

        def _scale(rbuf, sbuf):
            def body(g):
                ev = ex_v[pl.ds(g * LANES, LANES)]
                for r in range(LANES):
                    row = g * LANES + r
                    scl = ev[r]
                    vals = [rbuf[row, pl.ds(q * LANES, LANES)] * scl
                            for q in range(H // LANES)]
                    for q in range(H // LANES):
                        sbuf[row, pl.ds(q * LANES, LANES)] = vals[q]
                    sbuf[row, pl.ds(H, LANES)] = (
                        jnp.zeros((LANES,), jnp.float32) + scl)
            plsc.parallel_loop(0, CH // LANES)(body)

        NTRI = (NCH - 2) // 3
        _fire(0, rows0, gsem0)

        def _step(i, t, j):
            cur, nxt = bufs[t], bufs[(t + 1) % 3]
            scur = sbufs[t]
            _wait_g(j, cur, gsems[t])
            _fire(j + 1, nxt, gsems[(t + 1) % 3])
            _expass(j)
            if i is None:
                _wait_s(scur, ssems[t])
            else:
                @pl.when(i > 0)
                def _():
                    _wait_s(scur, ssems[t])
            _scale(cur, scur)
            _fire_s(j, scur, ssems[t])

        def _triple(i, carry):
            for t in range(3):
                _step(i, t, 3 * i + t)
            return carry
        lax.fori_loop(0, NTRI, _triple, None)
        jt = 3 * NTRI
        _step(None, 0, jt)
        _wait_g(jt + 1, rows1, gsem1)
        _expass(jt + 1)
        _wait_s(sb1, ssem1)
        _scale(rows1, sb1)
        _fire_s(jt + 1, sb1, ssem1)
        _wait_s(sb0, ssem0)
        _wait_s(sb1, ssem1)
        _wait_s(sb2, ssem2)

        plsc.subcore_barrier()
        pltpu.sync_copy(acc_sh.at[pl.ds(s * RPT, RPT)],
                        acc_out.at[(c * NH + k) * NS + s])

        @pl.when(k < NH - 1)
        def _():
            _wipe_acc()
        return carry

    lax.fori_loop(0, NH, _head, None)


def _phase2(h_all, src2, dst2, e_flat):
    mesh = plsc.VectorSubcoreMesh(core_axis_name="c", subcore_axis_name="s",
                                  num_cores=NC, num_subcores=NS)
    return pl.kernel(
        _sc_body,
        out_type=[
            jax.ShapeDtypeStruct((NC * NH * NS, RPT, HW2), jnp.float32),
        ],
        mesh=mesh,
        compiler_params=pltpu.CompilerParams(needs_layout_passes=False,
                                             use_tc_tiling_on_sc=False),
        scratch_types=[
            pltpu.VMEM((NCH, CH), jnp.int32),
            pltpu.VMEM((NCH, CH), jnp.int32),
            pltpu.VMEM((N,), jnp.float32),
            pltpu.VMEM((N,), jnp.float32),
            pltpu.VMEM((CH, H), jnp.float32),
            pltpu.VMEM((CH, H), jnp.float32),
            pltpu.VMEM((CH, H), jnp.float32),
            pltpu.VMEM((CH, HW2), jnp.float32),
            pltpu.VMEM((CH, HW2), jnp.float32),
            pltpu.VMEM((CH, HW2), jnp.float32),
            pltpu.VMEM((CH,), jnp.float32),
            pltpu.VMEM((ZR, HW2), jnp.float32),
            pltpu.VMEM_SHARED((NPAD, HW2), jnp.float32),
            pltpu.SemaphoreType.DMA,
            pltpu.SemaphoreType.DMA,
            pltpu.SemaphoreType.DMA,
            pltpu.SemaphoreType.DMA,
            pltpu.SemaphoreType.DMA,
            pltpu.SemaphoreType.DMA,
        ],
    )(h_all, src2, dst2, e_flat)


R3 = 1000


def _p3_body(acc_ref, x_ref, wh_ref, bh_ref, wl_ref, out_ref):
    a = acc_ref[...]
    asum = a[0] + a[1]
    parts = []
    for k in range(NH):
        g = asum[k, :, :H] / (asum[k, :, H][:, None] + 1e-16)
        parts.append(jnp.where(g > 0, g, jnp.exp(jnp.minimum(g, 0.0)) - 1.0))
    hcat = jnp.concatenate(parts, axis=1)
    z = jnp.dot(hcat, wh_ref[...], preferred_element_type=jnp.float32)
    z = z + bh_ref[...]
    z = jnp.dot(z, wl_ref[...], preferred_element_type=jnp.float32)
    out_ref[...] = z + x_ref[...]


def _phase3(acc, x, w_hidden, b_hidden, w_lin):
    return pl.pallas_call(
        _p3_body,
        grid=(N // R3,),
        in_specs=[
            pl.BlockSpec((NC, NH, R3, HW2), lambda i: (0, 0, i, 0)),
            pl.BlockSpec((R3, D), lambda i: (i, 0)),
            pl.BlockSpec((NH * H, LIN), lambda i: (0, 0)),
            pl.BlockSpec((1, LIN), lambda i: (0, 0)),
            pl.BlockSpec((LIN, OUT), lambda i: (0, 0)),
        ],
        out_specs=pl.BlockSpec((R3, OUT), lambda i: (i, 0)),
        out_shape=jax.ShapeDtypeStruct((N, OUT), jnp.float32),
    )(acc, x, w_hidden, b_hidden, w_lin)


def kernel(x, edge_index, W, a, W_hidden, b_hidden, W_lin):
    w_cat = jnp.transpose(W, (1, 0, 2)).reshape(D, NH * H)
    a2 = a.reshape(NH, 2, H)
    src2 = edge_index[0].reshape(E // CH, CH)
    dst2 = edge_index[1].reshape(E // CH, CH)
    h_all, e_arr = _phase1(x, w_cat, a2)
    acc_f, = _phase2(h_all, src2, dst2, e_arr.reshape(-1))
    acc = acc_f.reshape(NC, NH, NPAD, HW2)
    return _phase3(acc, x, W_hidden, b_hidden.reshape(1, LIN), W_lin)
